# Initial kernel scaffold; baseline (speedup 1.0000x reference)
import jax
import jax.numpy as jnp
from jax.experimental import pallas as pl


def kernel(x, weight):
    raise NotImplementedError("write your pallas kernel here")


# speedup vs baseline: 1.4741x; 1.4741x over previous
import functools
import math

import jax
import jax.numpy as jnp
from jax import lax
from jax.experimental import pallas as pl
from jax.experimental.pallas import tpu as pltpu
from jax.experimental.pallas import tpu_sc as plsc

DIM = 32
SCALE = math.sqrt(float(DIM))
LANES = 16

NC = 2
NS = 16
NW = NC * NS

B_TOTAL = 4096 * 200
B_PER_W = B_TOTAL // NW
CHUNK = 1024
N_CHUNKS = B_PER_W // CHUNK
SUB = 128
N_SUB = CHUNK // SUB

_mesh = plsc.VectorSubcoreMesh(core_axis_name="c", subcore_axis_name="s")


@functools.partial(
    pl.kernel,
    out_type=jax.ShapeDtypeStruct((B_TOTAL, DIM), jnp.float32),
    mesh=_mesh,
    scratch_types=[
        pltpu.VMEM((B_PER_W,), jnp.int32),
        pltpu.VMEM((CHUNK, DIM), jnp.float32),
        pltpu.VMEM((CHUNK, DIM), jnp.float32),
        pltpu.SemaphoreType.DMA,
        pltpu.SemaphoreType.DMA,
        pltpu.SemaphoreType.DMA,
        pltpu.SemaphoreType.DMA,
    ],
    compiler_params=pltpu.CompilerParams(use_tc_tiling_on_sc=False),
)
def _emb_lookup(x_hbm, w_hbm, out_hbm, idx_v, rows0, rows1, g0, g1, o0, o1):
    wid = lax.axis_index("s") * NC + lax.axis_index("c")
    base = wid * B_PER_W

    pltpu.sync_copy(x_hbm.at[pl.ds(base, B_PER_W)], idx_v)

    rows = (rows0, rows1)
    gsem = (g0, g1)
    osem = (o0, o1)

    def start_gather(g, b):
        off = g * CHUNK
        descs = []
        for s in range(N_SUB):
            descs.append(
                pltpu.async_copy(
                    w_hbm.at[idx_v.at[pl.ds(off + s * SUB, SUB)]],
                    rows[b].at[pl.ds(s * SUB, SUB)],
                    gsem[b],
                )
            )
        return descs

    def scale_chunk(r):
        def body(i, carry):
            r[i, pl.ds(0, LANES)] = r[i, pl.ds(0, LANES)] * SCALE
            r[i, pl.ds(LANES, LANES)] = r[i, pl.ds(LANES, LANES)] * SCALE
            return carry

        lax.fori_loop(0, CHUNK, body, 0, unroll=8)

    pend_out = [None, None]
    pend_g = [None, None]
    pend_g[0] = start_gather(0, 0)

    for g in range(N_CHUNKS):
        b = g & 1
        nb = b ^ 1
        if g + 1 < N_CHUNKS:
            if pend_out[nb] is not None:
                pend_out[nb].wait()
                pend_out[nb] = None
            pend_g[nb] = start_gather(g + 1, nb)
        for d in pend_g[b]:
            d.wait()
        scale_chunk(rows[b])
        pend_out[b] = pltpu.async_copy(
            rows[b], out_hbm.at[pl.ds(base + g * CHUNK, CHUNK)], osem[b]
        )

    for b in range(2):
        if pend_out[b] is not None:
            pend_out[b].wait()


def kernel(x, weight):
    xf = x.reshape(-1).astype(jnp.int32)
    out = _emb_lookup(xf, weight)
    return out.reshape(x.shape + (DIM,))


# speedup vs baseline: 1.8754x; 1.2722x over previous
import functools
import math

import jax
import jax.numpy as jnp
from jax import lax
from jax.experimental import pallas as pl
from jax.experimental.pallas import tpu as pltpu
from jax.experimental.pallas import tpu_sc as plsc

DIM = 32
SCALE = math.sqrt(float(DIM))
LANES = 16

NC = 2
NS = 16
NW = NC * NS

NSEQ = 4096
SEQLEN = 200
B_TOTAL = NSEQ * SEQLEN
B_PER_W = B_TOTAL // NW
SEQ_PER_W = NSEQ // NW
CB = 8
CHUNK = CB * SEQLEN
N_CHUNKS = SEQ_PER_W // CB
SUBS = [128] * 12 + [64]

_mesh = plsc.VectorSubcoreMesh(core_axis_name="c", subcore_axis_name="s")

_VPAD = 1 << 20
_QPAD = _VPAD // 4
_TP_BQ = 2048
_TP_GRID = _QPAD // _TP_BQ
_QUARTER_BLKS = _QPAD // _TP_BQ


def _retile_body(w0_ref, w1_ref, w2_ref, w3_ref, out_ref):
    parts = [ref[...].T for ref in (w0_ref, w1_ref, w2_ref, w3_ref)]
    out_ref[...] = jnp.concatenate(parts, axis=1)


_LAST_REAL_BLK = (1000000 - 1) // _TP_BQ


def _retile(wt):
    specs = [
        pl.BlockSpec(
            (32, _TP_BQ),
            functools.partial(
                lambda k, i: (0, jnp.minimum(k * _QUARTER_BLKS + i, _LAST_REAL_BLK)), k
            ),
        )
        for k in range(4)
    ]
    return pl.pallas_call(
        _retile_body,
        out_shape=jax.ShapeDtypeStruct((_QPAD, 128), jnp.float32),
        grid=(_TP_GRID,),
        in_specs=specs,
        out_specs=pl.BlockSpec((_TP_BQ, 128), lambda i: (i, 0)),
    )(wt, wt, wt, wt)


@functools.partial(
    pl.kernel,
    out_type=jax.ShapeDtypeStruct((NSEQ, SEQLEN, DIM), jnp.float32),
    mesh=_mesh,
    scratch_types=[
        pltpu.VMEM((B_PER_W,), jnp.int32),
        pltpu.VMEM((CHUNK, DIM), jnp.float32),
        pltpu.VMEM((CHUNK, DIM), jnp.float32),
        pltpu.SemaphoreType.DMA,
        pltpu.SemaphoreType.DMA,
        pltpu.SemaphoreType.DMA,
        pltpu.SemaphoreType.DMA,
    ],
    compiler_params=pltpu.CompilerParams(use_tc_tiling_on_sc=False),
)
def _emb_lookup(x_hbm, w_hbm, out3_hbm, idx_v, rows0, rows1, g0, g1, o0, o1):
    wid = lax.axis_index("s") * NC + lax.axis_index("c")
    base = wid * B_PER_W
    seq_base = wid * SEQ_PER_W

    pltpu.sync_copy(x_hbm.at[pl.ds(base, B_PER_W)], idx_v)

    def remap_body(j, carry):
        r = idx_v[pl.ds(j * LANES, LANES)]
        idx_v[pl.ds(j * LANES, LANES)] = ((r & (_QPAD - 1)) << 2) | (r >> 18)
        return carry

    lax.fori_loop(0, B_PER_W // LANES, remap_body, 0, unroll=8)

    rows = (rows0, rows1)
    gsem = (g0, g1)
    osem = (o0, o1)

    def start_gather(g, b):
        off = g * CHUNK
        descs = []
        pos = 0
        for sub in SUBS:
            descs.append(
                pltpu.async_copy(
                    w_hbm.at[idx_v.at[pl.ds(off + pos, sub)]],
                    rows[b].at[pl.ds(pos, sub)],
                    gsem[b],
                )
            )
            pos += sub
        return descs

    def scale_chunk(r):
        def body(i, carry):
            r[i, pl.ds(0, LANES)] = r[i, pl.ds(0, LANES)] * SCALE
            r[i, pl.ds(LANES, LANES)] = r[i, pl.ds(LANES, LANES)] * SCALE
            return carry

        lax.fori_loop(0, CHUNK, body, 0, unroll=8)

    pend_out = [None, None]
    pend_g = [None, None]
    pend_g[0] = start_gather(0, 0)

    for g in range(N_CHUNKS):
        b = g & 1
        nb = b ^ 1
        if g + 1 < N_CHUNKS:
            if pend_out[nb] is not None:
                for d in pend_out[nb]:
                    d.wait()
                pend_out[nb] = None
            pend_g[nb] = start_gather(g + 1, nb)
        for d in pend_g[b]:
            d.wait()
        scale_chunk(rows[b])
        pend_out[b] = [
            pltpu.async_copy(
                rows[b].at[pl.ds(k * SEQLEN, SEQLEN)],
                out3_hbm.at[seq_base + g * CB + k],
                osem[b],
            )
            for k in range(CB)
        ]

    for b in range(2):
        if pend_out[b] is not None:
            for d in pend_out[b]:
                d.wait()


def kernel(x, weight):
    xf = x.reshape(-1).astype(jnp.int32)
    w_lin = _retile(weight.T).reshape(_VPAD, DIM)
    return _emb_lookup(xf, w_lin)


# speedup vs baseline: 4.5148x; 2.4073x over previous
import functools
import math

import jax
import jax.numpy as jnp
from jax import lax
from jax.experimental import pallas as pl
from jax.experimental.pallas import tpu as pltpu
from jax.experimental.pallas import tpu_sc as plsc

DIM = 32
SCALE = math.sqrt(float(DIM))
LANES = 16

NC = 2
NS = 16
NW = NC * NS

NSEQ = 4096
SEQLEN = 200
B_TOTAL = NSEQ * SEQLEN
N_HALF = 2
H_SEQ = NSEQ // N_HALF
H_B = H_SEQ * SEQLEN
B_PER_W = H_B // NW
SEQ_PER_W = H_SEQ // NW
CB = 8
CHUNK = CB * SEQLEN
N_CHUNKS = SEQ_PER_W // CB
SUBS = [128] * 12 + [64]

_mesh = plsc.VectorSubcoreMesh(core_axis_name="c", subcore_axis_name="s")

_OB = 128
_OUT_GRID = H_SEQ // _OB


def _outpack_first_body(g_ref, out_ref):
    g3 = g_ref[...].reshape(_OB, 50, 128)
    h = jnp.transpose(g3, (1, 0, 2))
    p = jnp.transpose(h, (0, 2, 1))
    p4 = p.reshape(50, 4, 32, _OB)
    out_ref[...] = p4.reshape(200, 32, _OB)


def _outpack_rest_body(g_ref, prev_ref, out_ref):
    del prev_ref
    _outpack_first_body(g_ref, out_ref)


def _outpack(g128, half, prev):
    g_spec = pl.BlockSpec((_OB * 50, 128), lambda i: (i, 0))
    out_spec = pl.BlockSpec(
        (200, 32, _OB), functools.partial(lambda h, i: (0, 0, h * _OUT_GRID + i), half)
    )
    out_shape = jax.ShapeDtypeStruct((200, 32, NSEQ), jnp.float32)
    if prev is None:
        return pl.pallas_call(
            _outpack_first_body,
            out_shape=out_shape,
            grid=(_OUT_GRID,),
            in_specs=[g_spec],
            out_specs=out_spec,
        )(g128)
    return pl.pallas_call(
        _outpack_rest_body,
        out_shape=out_shape,
        grid=(_OUT_GRID,),
        in_specs=[g_spec, pl.BlockSpec(memory_space=pl.ANY)],
        out_specs=out_spec,
        input_output_aliases={1: 0},
    )(g128, prev)


_VPAD = 1 << 20
_QPAD = _VPAD // 4
_TP_BQ = 2048
_TP_GRID = _QPAD // _TP_BQ
_QUARTER_BLKS = _QPAD // _TP_BQ


def _retile_body(w0_ref, w1_ref, w2_ref, w3_ref, out_ref):
    stack = jnp.concatenate(
        [ref[...] for ref in (w0_ref, w1_ref, w2_ref, w3_ref)], axis=0
    )
    out_ref[...] = stack.T


_LAST_REAL_BLK = (1000000 - 1) // _TP_BQ


def _retile(wt):
    specs = [
        pl.BlockSpec(
            (32, _TP_BQ),
            functools.partial(
                lambda k, i: (0, jnp.minimum(k * _QUARTER_BLKS + i, _LAST_REAL_BLK)), k
            ),
        )
        for k in range(4)
    ]
    return pl.pallas_call(
        _retile_body,
        out_shape=jax.ShapeDtypeStruct((_QPAD, 128), jnp.float32),
        grid=(_TP_GRID,),
        in_specs=specs,
        out_specs=pl.BlockSpec((_TP_BQ, 128), lambda i: (i, 0)),
    )(wt, wt, wt, wt)


def _make_gather(half):
    @functools.partial(
        pl.kernel,
        out_type=jax.ShapeDtypeStruct((H_SEQ, SEQLEN, DIM), jnp.float32),
        mesh=_mesh,
        scratch_types=[
            pltpu.VMEM((B_PER_W,), jnp.int32),
            pltpu.VMEM((CHUNK, DIM), jnp.float32),
            pltpu.VMEM((CHUNK, DIM), jnp.float32),
            pltpu.SemaphoreType.DMA,
            pltpu.SemaphoreType.DMA,
            pltpu.SemaphoreType.DMA,
            pltpu.SemaphoreType.DMA,
        ],
        compiler_params=pltpu.CompilerParams(use_tc_tiling_on_sc=False),
    )
    def _gather_half(x_hbm, w_hbm, out3_hbm, idx_v, rows0, rows1, g0, g1, o0, o1):
        wid = lax.axis_index("s") * NC + lax.axis_index("c")
        base = half * H_B + wid * B_PER_W
        seq_base = wid * SEQ_PER_W

        pltpu.sync_copy(x_hbm.at[pl.ds(base, B_PER_W)], idx_v)

        def remap_body(j, carry):
            r = idx_v[pl.ds(j * LANES, LANES)]
            idx_v[pl.ds(j * LANES, LANES)] = ((r & (_QPAD - 1)) << 2) | (r >> 18)
            return carry

        lax.fori_loop(0, B_PER_W // LANES, remap_body, 0, unroll=8)

        rows = (rows0, rows1)
        gsem = (g0, g1)
        osem = (o0, o1)

        def start_gather(g, b):
            off = g * CHUNK
            descs = []
            pos = 0
            for sub in SUBS:
                descs.append(
                    pltpu.async_copy(
                        w_hbm.at[idx_v.at[pl.ds(off + pos, sub)]],
                        rows[b].at[pl.ds(pos, sub)],
                        gsem[b],
                    )
                )
                pos += sub
            return descs

        def scale_chunk(r):
            def body(i, carry):
                r[i, pl.ds(0, LANES)] = r[i, pl.ds(0, LANES)] * SCALE
                r[i, pl.ds(LANES, LANES)] = r[i, pl.ds(LANES, LANES)] * SCALE
                return carry

            lax.fori_loop(0, CHUNK, body, 0, unroll=8)

        pend_out = [None, None]
        pend_g = [None, None]
        pend_g[0] = start_gather(0, 0)

        for g in range(N_CHUNKS):
            b = g & 1
            nb = b ^ 1
            if g + 1 < N_CHUNKS:
                if pend_out[nb] is not None:
                    for d in pend_out[nb]:
                        d.wait()
                    pend_out[nb] = None
                pend_g[nb] = start_gather(g + 1, nb)
            for d in pend_g[b]:
                d.wait()
            scale_chunk(rows[b])
            pend_out[b] = [
                pltpu.async_copy(
                    rows[b].at[pl.ds(k * SEQLEN, SEQLEN)],
                    out3_hbm.at[seq_base + g * CB + k],
                    osem[b],
                )
                for k in range(CB)
            ]

        for b in range(2):
            if pend_out[b] is not None:
                for d in pend_out[b]:
                    d.wait()

    return _gather_half


_gather_halves = [_make_gather(h) for h in range(N_HALF)]


def kernel(x, weight):
    xf = x.reshape(-1).astype(jnp.int32)
    w_lin = _retile(weight.T).reshape(_VPAD, DIM)
    out = None
    for h in range(N_HALF):
        g = _gather_halves[h](xf, w_lin)
        g128 = g.reshape(H_B // 4, 128)
        out = _outpack(g128, h, out)
    return out.transpose(2, 0, 1)


# speedup vs baseline: 5.1075x; 1.1313x over previous
import functools
import math

import jax
import jax.numpy as jnp
from jax import lax
from jax.experimental import pallas as pl
from jax.experimental.pallas import tpu as pltpu
from jax.experimental.pallas import tpu_sc as plsc

DIM = 32
SCALE = math.sqrt(float(DIM))
LANES = 16

NC = 2
NS = 16
NW = NC * NS

NSEQ = 4096
SEQLEN = 200
B_TOTAL = NSEQ * SEQLEN
N_HALF = 2
H_SEQ = NSEQ // N_HALF
H_B = H_SEQ * SEQLEN
B_PER_W = H_B // NW
SEQ_PER_W = H_SEQ // NW
CB = 8
CHUNK = CB * SEQLEN
N_CHUNKS = SEQ_PER_W // CB
SUBS = [128] * 12 + [64]

_mesh = plsc.VectorSubcoreMesh(core_axis_name="c", subcore_axis_name="s")

_OB = 128
_OUT_GRID = H_SEQ // _OB


def _outpack_first_body(g_ref, out_ref):
    g3 = g_ref[...].reshape(_OB, 50, 128)
    h = jnp.transpose(g3, (1, 0, 2))
    p = jnp.transpose(h, (0, 2, 1))
    p4 = p.reshape(50, 4, 32, _OB)
    out_ref[...] = p4.reshape(200, 32, _OB)


def _outpack_rest_body(g_ref, prev_ref, out_ref):
    del prev_ref
    _outpack_first_body(g_ref, out_ref)


def _outpack(g128, half, prev):
    g_spec = pl.BlockSpec((_OB * 50, 128), lambda i: (i, 0))
    out_spec = pl.BlockSpec(
        (200, 32, _OB), functools.partial(lambda h, i: (0, 0, h * _OUT_GRID + i), half)
    )
    out_shape = jax.ShapeDtypeStruct((200, 32, NSEQ), jnp.float32)
    if prev is None:
        return pl.pallas_call(
            _outpack_first_body,
            out_shape=out_shape,
            grid=(_OUT_GRID,),
            in_specs=[g_spec],
            out_specs=out_spec,
        )(g128)
    return pl.pallas_call(
        _outpack_rest_body,
        out_shape=out_shape,
        grid=(_OUT_GRID,),
        in_specs=[g_spec, pl.BlockSpec(memory_space=pl.ANY)],
        out_specs=out_spec,
        input_output_aliases={1: 0},
    )(g128, prev)


_VPAD = 1 << 20
_QPAD = _VPAD // 4
_TP_BQ = 4096
_TP_GRID = _QPAD // _TP_BQ
_QUARTER_BLKS = _QPAD // _TP_BQ


def _retile_body(w0_ref, w1_ref, w2_ref, w3_ref, out_ref):
    stack = jnp.concatenate(
        [ref[...] for ref in (w0_ref, w1_ref, w2_ref, w3_ref)], axis=0
    )
    out_ref[...] = stack.T


_LAST_REAL_BLK = (1000000 - 1) // _TP_BQ


def _retile(wt):
    specs = [
        pl.BlockSpec(
            (32, _TP_BQ),
            functools.partial(
                lambda k, i: (0, jnp.minimum(k * _QUARTER_BLKS + i, _LAST_REAL_BLK)), k
            ),
        )
        for k in range(4)
    ]
    return pl.pallas_call(
        _retile_body,
        out_shape=jax.ShapeDtypeStruct((_QPAD, 128), jnp.float32),
        grid=(_TP_GRID,),
        in_specs=specs,
        out_specs=pl.BlockSpec((_TP_BQ, 128), lambda i: (i, 0)),
    )(wt, wt, wt, wt)


def _make_gather(half):
    @functools.partial(
        pl.kernel,
        out_type=jax.ShapeDtypeStruct((H_SEQ, SEQLEN, DIM), jnp.float32),
        mesh=_mesh,
        scratch_types=[
            pltpu.VMEM((B_PER_W,), jnp.int32),
            pltpu.VMEM((CHUNK, DIM), jnp.float32),
            pltpu.VMEM((CHUNK, DIM), jnp.float32),
            pltpu.SemaphoreType.DMA,
            pltpu.SemaphoreType.DMA,
            pltpu.SemaphoreType.DMA,
            pltpu.SemaphoreType.DMA,
        ],
        compiler_params=pltpu.CompilerParams(use_tc_tiling_on_sc=False),
    )
    def _gather_half(x_hbm, w_hbm, out3_hbm, idx_v, rows0, rows1, g0, g1, o0, o1):
        wid = lax.axis_index("s") * NC + lax.axis_index("c")
        base = half * H_B + wid * B_PER_W
        seq_base = wid * SEQ_PER_W

        pltpu.sync_copy(x_hbm.at[pl.ds(base, B_PER_W)], idx_v)

        def remap_body(j, carry):
            r = idx_v[pl.ds(j * LANES, LANES)]
            idx_v[pl.ds(j * LANES, LANES)] = ((r & (_QPAD - 1)) << 2) | (r >> 18)
            return carry

        lax.fori_loop(0, B_PER_W // LANES, remap_body, 0, unroll=8)

        rows = (rows0, rows1)
        gsem = (g0, g1)
        osem = (o0, o1)

        def start_gather(g, b):
            off = g * CHUNK
            descs = []
            pos = 0
            for sub in SUBS:
                descs.append(
                    pltpu.async_copy(
                        w_hbm.at[idx_v.at[pl.ds(off + pos, sub)]],
                        rows[b].at[pl.ds(pos, sub)],
                        gsem[b],
                    )
                )
                pos += sub
            return descs

        def scale_chunk(r):
            def body(i, carry):
                r[i, pl.ds(0, LANES)] = r[i, pl.ds(0, LANES)] * SCALE
                r[i, pl.ds(LANES, LANES)] = r[i, pl.ds(LANES, LANES)] * SCALE
                return carry

            lax.fori_loop(0, CHUNK, body, 0, unroll=8)

        pend_out = [None, None]
        pend_g = [None, None]
        pend_g[0] = start_gather(0, 0)

        for g in range(N_CHUNKS):
            b = g & 1
            nb = b ^ 1
            if g + 1 < N_CHUNKS:
                if pend_out[nb] is not None:
                    for d in pend_out[nb]:
                        d.wait()
                    pend_out[nb] = None
                pend_g[nb] = start_gather(g + 1, nb)
            for d in pend_g[b]:
                d.wait()
            scale_chunk(rows[b])
            pend_out[b] = [
                pltpu.async_copy(
                    rows[b].at[pl.ds(k * SEQLEN, SEQLEN)],
                    out3_hbm.at[seq_base + g * CB + k],
                    osem[b],
                )
                for k in range(CB)
            ]

        for b in range(2):
            if pend_out[b] is not None:
                for d in pend_out[b]:
                    d.wait()

    return _gather_half


_gather_halves = [_make_gather(h) for h in range(N_HALF)]


def kernel(x, weight):
    xf = x.reshape(-1).astype(jnp.int32)
    w_lin = _retile(weight.T).reshape(_VPAD, DIM)
    out = None
    for h in range(N_HALF):
        g = _gather_halves[h](xf, w_lin)
        g128 = g.reshape(H_B // 4, 128)
        out = _outpack(g128, h, out)
    return out.transpose(2, 0, 1)


# speedup vs baseline: 5.4174x; 1.0607x over previous
import functools
import math

import jax
import jax.numpy as jnp
from jax import lax
from jax.experimental import pallas as pl
from jax.experimental.pallas import tpu as pltpu
from jax.experimental.pallas import tpu_sc as plsc

DIM = 32
SCALE = math.sqrt(float(DIM))
LANES = 16

NC = 2
NS = 16
NW = NC * NS

NSEQ = 4096
SEQLEN = 200
B_TOTAL = NSEQ * SEQLEN
N_HALF = 2
H_SEQ = NSEQ // N_HALF
H_B = H_SEQ * SEQLEN
B_PER_W = H_B // NW
SEQ_PER_W = H_SEQ // NW
CB = 8
CHUNK = CB * SEQLEN
N_CHUNKS = SEQ_PER_W // CB
SUBS = [128] * 12 + [64]

_mesh = plsc.VectorSubcoreMesh(core_axis_name="c", subcore_axis_name="s")

_OB = 128
_OUT_GRID = H_SEQ // _OB


def _outpack_first_body(g_ref, out_ref):
    g3 = g_ref[...].reshape(_OB, 50, 128)
    h = jnp.transpose(g3, (1, 0, 2))
    p = jnp.transpose(h, (0, 2, 1))
    p4 = p.reshape(50, 4, 32, _OB)
    out_ref[...] = p4.reshape(200, 32, _OB)


def _outpack_rest_body(g_ref, prev_ref, out_ref):
    del prev_ref
    _outpack_first_body(g_ref, out_ref)


def _outpack(g128, half, prev):
    g_spec = pl.BlockSpec((_OB * 50, 128), lambda i: (i, 0))
    out_spec = pl.BlockSpec(
        (200, 32, _OB), functools.partial(lambda h, i: (0, 0, h * _OUT_GRID + i), half)
    )
    out_shape = jax.ShapeDtypeStruct((200, 32, NSEQ), jnp.float32)
    if prev is None:
        return pl.pallas_call(
            _outpack_first_body,
            out_shape=out_shape,
            grid=(_OUT_GRID,),
            in_specs=[g_spec],
            out_specs=out_spec,
        )(g128)
    return pl.pallas_call(
        _outpack_rest_body,
        out_shape=out_shape,
        grid=(_OUT_GRID,),
        in_specs=[g_spec, pl.BlockSpec(memory_space=pl.ANY)],
        out_specs=out_spec,
        input_output_aliases={1: 0},
    )(g128, prev)


_VPAD = 1 << 20
_QPAD = _VPAD // 4
_TP_BQ = 8192
_TP_GRID = _QPAD // _TP_BQ
_QUARTER_BLKS = _QPAD // _TP_BQ


def _retile_body(w0_ref, w1_ref, w2_ref, w3_ref, out_ref):
    stack = jnp.concatenate(
        [ref[...] for ref in (w0_ref, w1_ref, w2_ref, w3_ref)], axis=0
    )
    out_ref[...] = stack.T


_LAST_REAL_BLK = (1000000 - 1) // _TP_BQ


def _retile(wt):
    specs = [
        pl.BlockSpec(
            (32, _TP_BQ),
            functools.partial(
                lambda k, i: (0, jnp.minimum(k * _QUARTER_BLKS + i, _LAST_REAL_BLK)), k
            ),
        )
        for k in range(4)
    ]
    return pl.pallas_call(
        _retile_body,
        out_shape=jax.ShapeDtypeStruct((_QPAD, 128), jnp.float32),
        grid=(_TP_GRID,),
        in_specs=specs,
        out_specs=pl.BlockSpec((_TP_BQ, 128), lambda i: (i, 0)),
    )(wt, wt, wt, wt)


def _make_gather(half):
    @functools.partial(
        pl.kernel,
        out_type=jax.ShapeDtypeStruct((H_SEQ, SEQLEN, DIM), jnp.float32),
        mesh=_mesh,
        scratch_types=[
            pltpu.VMEM((B_PER_W,), jnp.int32),
            pltpu.VMEM((CHUNK, DIM), jnp.float32),
            pltpu.VMEM((CHUNK, DIM), jnp.float32),
            pltpu.SemaphoreType.DMA,
            pltpu.SemaphoreType.DMA,
            pltpu.SemaphoreType.DMA,
            pltpu.SemaphoreType.DMA,
        ],
        compiler_params=pltpu.CompilerParams(use_tc_tiling_on_sc=False),
    )
    def _gather_half(x_hbm, w_hbm, out3_hbm, idx_v, rows0, rows1, g0, g1, o0, o1):
        wid = lax.axis_index("s") * NC + lax.axis_index("c")
        base = half * H_B + wid * B_PER_W
        seq_base = wid * SEQ_PER_W

        pltpu.sync_copy(x_hbm.at[pl.ds(base, B_PER_W)], idx_v)

        def remap_body(j, carry):
            r = idx_v[pl.ds(j * LANES, LANES)]
            idx_v[pl.ds(j * LANES, LANES)] = ((r & (_QPAD - 1)) << 2) | (r >> 18)
            return carry

        lax.fori_loop(0, B_PER_W // LANES, remap_body, 0, unroll=8)

        rows = (rows0, rows1)
        gsem = (g0, g1)
        osem = (o0, o1)

        def start_gather(g, b):
            off = g * CHUNK
            descs = []
            pos = 0
            for sub in SUBS:
                descs.append(
                    pltpu.async_copy(
                        w_hbm.at[idx_v.at[pl.ds(off + pos, sub)]],
                        rows[b].at[pl.ds(pos, sub)],
                        gsem[b],
                    )
                )
                pos += sub
            return descs

        def scale_chunk(r):
            def body(i, carry):
                r[i, pl.ds(0, LANES)] = r[i, pl.ds(0, LANES)] * SCALE
                r[i, pl.ds(LANES, LANES)] = r[i, pl.ds(LANES, LANES)] * SCALE
                return carry

            lax.fori_loop(0, CHUNK, body, 0, unroll=8)

        pend_out = [None, None]
        pend_g = [None, None]
        pend_g[0] = start_gather(0, 0)

        for g in range(N_CHUNKS):
            b = g & 1
            nb = b ^ 1
            if g + 1 < N_CHUNKS:
                if pend_out[nb] is not None:
                    for d in pend_out[nb]:
                        d.wait()
                    pend_out[nb] = None
                pend_g[nb] = start_gather(g + 1, nb)
            for d in pend_g[b]:
                d.wait()
            scale_chunk(rows[b])
            pend_out[b] = [
                pltpu.async_copy(
                    rows[b].at[pl.ds(k * SEQLEN, SEQLEN)],
                    out3_hbm.at[seq_base + g * CB + k],
                    osem[b],
                )
                for k in range(CB)
            ]

        for b in range(2):
            if pend_out[b] is not None:
                for d in pend_out[b]:
                    d.wait()

    return _gather_half


_gather_halves = [_make_gather(h) for h in range(N_HALF)]


def kernel(x, weight):
    xf = x.reshape(-1).astype(jnp.int32)
    w_lin = _retile(weight.T).reshape(_VPAD, DIM)
    out = None
    for h in range(N_HALF):
        g = _gather_halves[h](xf, w_lin)
        g128 = g.reshape(H_B // 4, 128)
        out = _outpack(g128, h, out)
    return out.transpose(2, 0, 1)


# speedup vs baseline: 5.4578x; 1.0075x over previous
import functools
import math

import jax
import jax.numpy as jnp
from jax import lax
from jax.experimental import pallas as pl
from jax.experimental.pallas import tpu as pltpu
from jax.experimental.pallas import tpu_sc as plsc

DIM = 32
SCALE = math.sqrt(float(DIM))
LANES = 16

NC = 2
NS = 16
NW = NC * NS

NSEQ = 4096
SEQLEN = 200
B_TOTAL = NSEQ * SEQLEN
N_HALF = 2
H_SEQ = NSEQ // N_HALF
H_B = H_SEQ * SEQLEN
B_PER_W = H_B // NW
SEQ_PER_W = H_SEQ // NW
CB = 8
CHUNK = CB * SEQLEN
N_CHUNKS = SEQ_PER_W // CB
SUBS = [128] * 12 + [64]

_mesh = plsc.VectorSubcoreMesh(core_axis_name="c", subcore_axis_name="s")

_OB = 128
_OUT_GRID = H_SEQ // _OB


def _outpack_first_body(g_ref, out_ref):
    g3 = g_ref[...].reshape(_OB, 50, 128)
    h = jnp.transpose(g3, (1, 0, 2))
    p = jnp.transpose(h, (0, 2, 1))
    p4 = p.reshape(50, 4, 32, _OB)
    out_ref[...] = p4.reshape(200, 32, _OB)


def _outpack_rest_body(g_ref, prev_ref, out_ref):
    del prev_ref
    _outpack_first_body(g_ref, out_ref)


def _outpack(g128, half, prev):
    g_spec = pl.BlockSpec((_OB * 50, 128), lambda i: (i, 0))
    out_spec = pl.BlockSpec(
        (200, 32, _OB), functools.partial(lambda h, i: (0, 0, h * _OUT_GRID + i), half)
    )
    out_shape = jax.ShapeDtypeStruct((200, 32, NSEQ), jnp.float32)
    if prev is None:
        return pl.pallas_call(
            _outpack_first_body,
            out_shape=out_shape,
            grid=(_OUT_GRID,),
            in_specs=[g_spec],
            out_specs=out_spec,
        )(g128)
    return pl.pallas_call(
        _outpack_rest_body,
        out_shape=out_shape,
        grid=(_OUT_GRID,),
        in_specs=[g_spec, pl.BlockSpec(memory_space=pl.ANY)],
        out_specs=out_spec,
        input_output_aliases={1: 0},
    )(g128, prev)


_VPAD = 1 << 20
_QPAD = _VPAD // 4
_TP_BQ = 16384
_TP_GRID = _QPAD // _TP_BQ
_QUARTER_BLKS = _QPAD // _TP_BQ


def _retile_body(w0_ref, w1_ref, w2_ref, w3_ref, out_ref):
    stack = jnp.concatenate(
        [ref[...] for ref in (w0_ref, w1_ref, w2_ref, w3_ref)], axis=0
    )
    out_ref[...] = stack.T


_LAST_REAL_BLK = (1000000 - 1) // _TP_BQ


def _retile(wt):
    specs = [
        pl.BlockSpec(
            (32, _TP_BQ),
            functools.partial(
                lambda k, i: (0, jnp.minimum(k * _QUARTER_BLKS + i, _LAST_REAL_BLK)), k
            ),
        )
        for k in range(4)
    ]
    return pl.pallas_call(
        _retile_body,
        out_shape=jax.ShapeDtypeStruct((_QPAD, 128), jnp.float32),
        grid=(_TP_GRID,),
        in_specs=specs,
        out_specs=pl.BlockSpec((_TP_BQ, 128), lambda i: (i, 0)),
    )(wt, wt, wt, wt)


def _make_gather(half):
    @functools.partial(
        pl.kernel,
        out_type=jax.ShapeDtypeStruct((H_SEQ, SEQLEN, DIM), jnp.float32),
        mesh=_mesh,
        scratch_types=[
            pltpu.VMEM((B_PER_W,), jnp.int32),
            pltpu.VMEM((CHUNK, DIM), jnp.float32),
            pltpu.VMEM((CHUNK, DIM), jnp.float32),
            pltpu.SemaphoreType.DMA,
            pltpu.SemaphoreType.DMA,
            pltpu.SemaphoreType.DMA,
            pltpu.SemaphoreType.DMA,
        ],
        compiler_params=pltpu.CompilerParams(use_tc_tiling_on_sc=False),
    )
    def _gather_half(x_hbm, w_hbm, out3_hbm, idx_v, rows0, rows1, g0, g1, o0, o1):
        wid = lax.axis_index("s") * NC + lax.axis_index("c")
        base = half * H_B + wid * B_PER_W
        seq_base = wid * SEQ_PER_W

        pltpu.sync_copy(x_hbm.at[pl.ds(base, B_PER_W)], idx_v)

        def remap_body(j, carry):
            r = idx_v[pl.ds(j * LANES, LANES)]
            idx_v[pl.ds(j * LANES, LANES)] = ((r & (_QPAD - 1)) << 2) | (r >> 18)
            return carry

        lax.fori_loop(0, B_PER_W // LANES, remap_body, 0, unroll=8)

        rows = (rows0, rows1)
        gsem = (g0, g1)
        osem = (o0, o1)

        def start_gather(g, b):
            off = g * CHUNK
            descs = []
            pos = 0
            for sub in SUBS:
                descs.append(
                    pltpu.async_copy(
                        w_hbm.at[idx_v.at[pl.ds(off + pos, sub)]],
                        rows[b].at[pl.ds(pos, sub)],
                        gsem[b],
                    )
                )
                pos += sub
            return descs

        def scale_chunk(r):
            def body(i, carry):
                r[i, pl.ds(0, LANES)] = r[i, pl.ds(0, LANES)] * SCALE
                r[i, pl.ds(LANES, LANES)] = r[i, pl.ds(LANES, LANES)] * SCALE
                return carry

            lax.fori_loop(0, CHUNK, body, 0, unroll=8)

        pend_out = [None, None]
        pend_g = [None, None]
        pend_g[0] = start_gather(0, 0)

        for g in range(N_CHUNKS):
            b = g & 1
            nb = b ^ 1
            if g + 1 < N_CHUNKS:
                if pend_out[nb] is not None:
                    for d in pend_out[nb]:
                        d.wait()
                    pend_out[nb] = None
                pend_g[nb] = start_gather(g + 1, nb)
            for d in pend_g[b]:
                d.wait()
            scale_chunk(rows[b])
            pend_out[b] = [
                pltpu.async_copy(
                    rows[b].at[pl.ds(k * SEQLEN, SEQLEN)],
                    out3_hbm.at[seq_base + g * CB + k],
                    osem[b],
                )
                for k in range(CB)
            ]

        for b in range(2):
            if pend_out[b] is not None:
                for d in pend_out[b]:
                    d.wait()

    return _gather_half


_gather_halves = [_make_gather(h) for h in range(N_HALF)]


def kernel(x, weight):
    xf = x.reshape(-1).astype(jnp.int32)
    w_lin = _retile(weight.T).reshape(_VPAD, DIM)
    out = None
    for h in range(N_HALF):
        g = _gather_halves[h](xf, w_lin)
        g128 = g.reshape(H_B // 4, 128)
        out = _outpack(g128, h, out)
    return out.transpose(2, 0, 1)
